# baseline (speedup 1.0000x reference)
import functools

import jax
import jax.numpy as jnp
from jax import lax
from jax.experimental import pallas as pl
from jax.experimental.pallas import tpu as pltpu
from jax.experimental.pallas import tpu_sc as plsc

BATCH = 16384
N_FIELDS = 26
D_EMB = 64
B_FLAT = BATCH * N_FIELDS

_NC = 2
_NS = 16
_NW = _NC * _NS
_B_PER_W = B_FLAT // _NW
_CHUNK = 128
_N_CHUNKS = _B_PER_W // _CHUNK

_mesh = plsc.VectorSubcoreMesh(core_axis_name="c", subcore_axis_name="s")


@functools.partial(
    pl.kernel,
    mesh=_mesh,
    out_type=jax.ShapeDtypeStruct((B_FLAT, D_EMB), jnp.float32),
    scratch_types=[
        pltpu.VMEM((_CHUNK,), jnp.int32),
        pltpu.VMEM((_CHUNK, D_EMB), jnp.float32),
        pltpu.SemaphoreType.DMA,
    ],
    compiler_params=pltpu.CompilerParams(use_tc_tiling_on_sc=False),
)
def _gather_sc(idx_hbm, table_hbm, out_hbm, idx_v, rows_v, sem):
    wid = lax.axis_index("s") * _NC + lax.axis_index("c")
    base = wid * _B_PER_W

    def body(i, carry):
        off = base + i * _CHUNK
        pltpu.sync_copy(idx_hbm.at[pl.ds(off, _CHUNK)], idx_v)
        pltpu.async_copy(table_hbm.at[idx_v], rows_v, sem).wait()
        pltpu.sync_copy(rows_v, out_hbm.at[pl.ds(off, _CHUNK)])
        return carry

    lax.fori_loop(0, _N_CHUNKS, body, 0)


def kernel(idx, table):
    idx_flat = idx.reshape(B_FLAT).astype(jnp.int32)
    out = _gather_sc(idx_flat, table)
    return out.reshape(BATCH, N_FIELDS, D_EMB)


# speedup vs baseline: 1.1298x; 1.1298x over previous
import functools

import jax
import jax.numpy as jnp
from jax import lax
from jax.experimental import pallas as pl
from jax.experimental.pallas import tpu as pltpu
from jax.experimental.pallas import tpu_sc as plsc

BATCH = 16384
N_FIELDS = 26
D_EMB = 64
B_FLAT = BATCH * N_FIELDS

_NC = 2
_NS = 16
_NW = _NC * _NS
_B_PER_W = B_FLAT // _NW
_CHUNK = 832
_N_CHUNKS = _B_PER_W // _CHUNK

_mesh = plsc.VectorSubcoreMesh(core_axis_name="c", subcore_axis_name="s")


@functools.partial(
    pl.kernel,
    mesh=_mesh,
    out_type=jax.ShapeDtypeStruct((B_FLAT, D_EMB), jnp.float32),
    scratch_types=[
        pltpu.VMEM((_N_CHUNKS, _CHUNK), jnp.int32),
        pltpu.VMEM((_CHUNK, D_EMB), jnp.float32),
        pltpu.VMEM((_CHUNK, D_EMB), jnp.float32),
        pltpu.SemaphoreType.DMA,
        pltpu.SemaphoreType.DMA,
        pltpu.SemaphoreType.DMA,
        pltpu.SemaphoreType.DMA,
    ],
    compiler_params=pltpu.CompilerParams(use_tc_tiling_on_sc=False),
)
def _gather_sc(idx_hbm, table_hbm, out_hbm, idx_v, rows0, rows1,
               gsem0, gsem1, osem0, osem1):
    wid = lax.axis_index("s") * _NC + lax.axis_index("c")
    base = wid * _B_PER_W
    pltpu.sync_copy(idx_hbm.at[wid], idx_v)

    rows = (rows0, rows1)
    gsem = (gsem0, gsem1)
    osem = (osem0, osem1)

    def start_gather(i):
        b = i % 2
        return pltpu.async_copy(table_hbm.at[idx_v.at[i]], rows[b], gsem[b])

    def start_out(i):
        b = i % 2
        return pltpu.async_copy(
            rows[b], out_hbm.at[pl.ds(base + i * _CHUNK, _CHUNK)], osem[b])

    g = [None] * _N_CHUNKS
    o = [None] * _N_CHUNKS
    g[0] = start_gather(0)
    g[1] = start_gather(1)
    for i in range(_N_CHUNKS):
        g[i].wait()
        o[i] = start_out(i)
        if i + 2 < _N_CHUNKS:
            o[i].wait()
            g[i + 2] = start_gather(i + 2)
    o[_N_CHUNKS - 2].wait()
    o[_N_CHUNKS - 1].wait()


def kernel(idx, table):
    idx_r = idx.reshape(_NW, _N_CHUNKS, _CHUNK).astype(jnp.int32)
    out = _gather_sc(idx_r, table)
    return out.reshape(BATCH, N_FIELDS, D_EMB)


# speedup vs baseline: 1.1327x; 1.0026x over previous
import functools

import jax
import jax.numpy as jnp
from jax import lax
from jax.experimental import pallas as pl
from jax.experimental.pallas import tpu as pltpu
from jax.experimental.pallas import tpu_sc as plsc

BATCH = 16384
N_FIELDS = 26
D_EMB = 64
N_ROWS = BATCH * N_FIELDS

_NC = 2
_NS = 16
_NW = _NC * _NS
_ROWS_PER_W = N_ROWS // _NW
_CROWS = 512
_N_CHUNKS = _ROWS_PER_W // _CROWS

_mesh = plsc.VectorSubcoreMesh(core_axis_name="c", subcore_axis_name="s")


@functools.partial(
    pl.kernel,
    mesh=_mesh,
    out_type=jax.ShapeDtypeStruct((N_ROWS, D_EMB), jnp.float32),
    scratch_types=[
        pltpu.VMEM((_ROWS_PER_W,), jnp.int32),
        pltpu.VMEM((_CROWS, D_EMB), jnp.float32),
        pltpu.VMEM((_CROWS, D_EMB), jnp.float32),
        pltpu.SemaphoreType.DMA,
        pltpu.SemaphoreType.DMA,
        pltpu.SemaphoreType.DMA,
        pltpu.SemaphoreType.DMA,
    ],
    compiler_params=pltpu.CompilerParams(use_tc_tiling_on_sc=False),
)
def _gather_sc(idx_hbm, table_hbm, out_hbm, idx_v, rows0, rows1,
               gsem0, gsem1, osem0, osem1):
    wid = lax.axis_index("s") * _NC + lax.axis_index("c")
    base = wid * _ROWS_PER_W
    pltpu.sync_copy(idx_hbm.at[pl.ds(base, _ROWS_PER_W)], idx_v)

    rows = (rows0, rows1)
    gsem = (gsem0, gsem1)
    osem = (osem0, osem1)

    def start_gather(i):
        b = i % 2
        return pltpu.async_copy(
            table_hbm.at[idx_v.at[pl.ds(i * _CROWS, _CROWS)]], rows[b], gsem[b])

    def start_out(i):
        b = i % 2
        return pltpu.async_copy(
            rows[b], out_hbm.at[pl.ds(base + i * _CROWS, _CROWS)], osem[b])

    g = [None] * _N_CHUNKS
    o = [None] * _N_CHUNKS
    g[0] = start_gather(0)
    g[1] = start_gather(1)
    for i in range(_N_CHUNKS):
        g[i].wait()
        o[i] = start_out(i)
        if i + 2 < _N_CHUNKS:
            o[i].wait()
            g[i + 2] = start_gather(i + 2)
    o[_N_CHUNKS - 2].wait()
    o[_N_CHUNKS - 1].wait()


def kernel(idx, table):
    flat = _gather_sc(idx.astype(jnp.int32).reshape(-1), table)
    return flat.reshape(BATCH, N_FIELDS, D_EMB)
